# Initial kernel scaffold; baseline (speedup 1.0000x reference)
import jax
import jax.numpy as jnp
from jax.experimental import pallas as pl


def kernel(x, edge_index, e_weight, W1, b1, W2, b2):
    raise NotImplementedError("write your pallas kernel here")


# speedup vs baseline: 7.1064x; 7.1064x over previous
import functools

import jax
import jax.numpy as jnp
from jax import lax
from jax.experimental import pallas as pl
from jax.experimental.pallas import tpu as pltpu
from jax.experimental.pallas import tpu_sc as plsc

N = 10000
E = 320000
D = 128

NC = 2
NS = 16
NW = NC * NS
EPW = E // NW
CW = 40
NCH = EPW // CW
NP = 10240
RPS = NP // NS
RZ = 128
HW = 16
LANES = 16

_mesh = plsc.VectorSubcoreMesh(core_axis_name="c", subcore_axis_name="s")
_sc_params = pltpu.CompilerParams(use_tc_tiling_on_sc=False,
                                  needs_layout_passes=False)


@functools.partial(
    pl.kernel,
    out_type=[jax.ShapeDtypeStruct((NC, NP, HW), jnp.float32),
              jax.ShapeDtypeStruct((NC, NP, HW), jnp.float32)],
    mesh=_mesh,
    scratch_types=[
        pltpu.VMEM((NCH, CW), jnp.int32),
        pltpu.VMEM((NCH, CW), jnp.int32),
        pltpu.VMEM((CW, HW), jnp.float32),
        pltpu.VMEM((RPS, HW), jnp.float32),
        pltpu.VMEM_SHARED((NP, HW), jnp.float32),
        pltpu.VMEM_SHARED((NP, HW), jnp.float32),
        pltpu.SemaphoreType.DMA,
        pltpu.SemaphoreType.DMA,
    ],
    compiler_params=_sc_params,
)
def _hist(srcC, dstC, outO, outI, sbuf, dbuf, ones, zbuf, accO, accI,
          sem0, sem1):
    c = lax.axis_index("c")
    s = lax.axis_index("s")
    w = c * NS + s

    pltpu.sync_copy(srcC.at[w], sbuf)
    pltpu.sync_copy(dstC.at[w], dbuf)

    @pl.loop(0, CW)
    def _(i):
        ones[i, :] = jnp.full((HW,), 1.0, jnp.float32)

    @pl.loop(0, RPS)
    def _(i):
        zbuf[i, :] = jnp.full((HW,), 0.0, jnp.float32)

    sl = pl.ds(s * RPS, RPS)
    pltpu.sync_copy(zbuf, accO.at[sl])
    pltpu.sync_copy(zbuf, accI.at[sl])
    plsc.subcore_barrier()

    @pl.loop(0, NCH, step=2)
    def _(j):
        d0 = pltpu.async_copy(ones, accO.at[sbuf.at[j]], sem0, add=True)
        d1 = pltpu.async_copy(ones, accI.at[dbuf.at[j]], sem0, add=True)
        d2 = pltpu.async_copy(ones, accO.at[sbuf.at[j + 1]], sem1, add=True)
        d3 = pltpu.async_copy(ones, accI.at[dbuf.at[j + 1]], sem1, add=True)
        d0.wait()
        d1.wait()
        d2.wait()
        d3.wait()

    plsc.subcore_barrier()
    pltpu.sync_copy(accO.at[sl], outO.at[c, sl])
    pltpu.sync_copy(accI.at[sl], outI.at[c, sl])


BCH = 50
NBLK = NCH // BCH
BEW = 25
ZR = 16


@functools.partial(
    pl.kernel,
    out_type=jax.ShapeDtypeStruct((NC, NP, D), jnp.float32),
    mesh=_mesh,
    scratch_types=[
        pltpu.VMEM((NCH, CW), jnp.int32),
        pltpu.VMEM((BCH, CW), jnp.int32),
        pltpu.VMEM((BCH, CW), jnp.int32),
        pltpu.VMEM((BEW, CW), jnp.float32),
        pltpu.VMEM((NCH, CW), jnp.float32),
        pltpu.VMEM((NP,), jnp.float32),
        pltpu.VMEM((CW, D), jnp.float32),
        pltpu.VMEM((CW, D), jnp.float32),
        pltpu.VMEM((ZR, D), jnp.float32),
        pltpu.VMEM_SHARED((NP, D), jnp.float32),
        pltpu.SemaphoreType.DMA,
        pltpu.SemaphoreType.DMA,
        pltpu.SemaphoreType.DMA,
    ],
    compiler_params=_sc_params,
)
def _agg(h, srcC, dstC, ewC, ns, out, sbuf, dblkA, dblkB, ewb, wpr, nbuf,
         rows0, rows1, zbuf, acc, gsem0, gsem1, dsem):
    c = lax.axis_index("c")
    s = lax.axis_index("s")
    w = c * NS + s

    pltpu.sync_copy(srcC.at[w], sbuf)
    pltpu.sync_copy(ns, nbuf)

    @pl.loop(0, ZR)
    def _(i):
        for k in range(D // LANES):
            zbuf[i, pl.ds(k * LANES, LANES)] = jnp.full(
                (LANES,), 0.0, jnp.float32)

    @pl.loop(0, RPS // ZR)
    def _(t):
        pltpu.sync_copy(zbuf, acc.at[pl.ds(s * RPS + t * ZR, ZR)])

    for bw in range(NCH // BEW):
        pltpu.sync_copy(ewC.at[w, pl.ds(bw * BEW, BEW)], ewb)

        @pl.loop(bw * BEW, (bw + 1) * BEW)
        def _(j):
            for col in (0, 16, 24):
                cs = pl.ds(col, LANES)
                idx = sbuf[j, cs]
                nv = plsc.load_gather(nbuf, [idx])
                wpr[j, cs] = ewb[j - bw * BEW, cs] * nv

    plsc.subcore_barrier()

    def scale(rbuf, j):
        for (col, lo) in ((0, 0), (16, 0), (24, 8)):
            w16 = wpr[j, pl.ds(col, LANES)]
            for t in range(lo, LANES):
                e = col + t
                wv = w16[t]
                for k in range(D // LANES):
                    csl = pl.ds(k * LANES, LANES)
                    rbuf[e, csl] = rbuf[e, csl] * wv

    dblks = (dblkA, dblkB)
    pltpu.sync_copy(dstC.at[w, pl.ds(0, BCH)], dblkA)
    pltpu.async_copy(h.at[sbuf.at[0]], rows0, gsem0)
    pltpu.async_copy(h.at[sbuf.at[1]], rows1, gsem1)

    for bo in range(NBLK):
        dcur = dblks[bo % 2]
        dnxt = dblks[(bo + 1) % 2]
        if bo + 1 < NBLK:
            dpref = pltpu.async_copy(
                dstC.at[w, pl.ds((bo + 1) * BCH, BCH)], dnxt, dsem)

        @pl.loop(0, BCH, step=2)
        def _(jj):
            j = bo * BCH + jj
            pltpu.make_async_copy(h.at[sbuf.at[j]], rows0, gsem0).wait()
            scale(rows0, j)
            pltpu.sync_copy(rows0, acc.at[dcur.at[jj]], add=True)

            @pl.when(j + 2 < NCH)
            def _():
                pltpu.async_copy(h.at[sbuf.at[j + 2]], rows0, gsem0)

            pltpu.make_async_copy(h.at[sbuf.at[j + 1]], rows1, gsem1).wait()
            scale(rows1, j + 1)
            pltpu.sync_copy(rows1, acc.at[dcur.at[jj + 1]], add=True)

            @pl.when(j + 3 < NCH)
            def _():
                pltpu.async_copy(h.at[sbuf.at[j + 3]], rows1, gsem1)

        if bo + 1 < NBLK:
            dpref.wait()

    plsc.subcore_barrier()

    @pl.loop(0, RPS // ZR)
    def _(t):
        osl = pl.ds(s * RPS + t * ZR, ZR)
        pltpu.sync_copy(acc.at[osl], out.at[c, osl])


_BLK = 1000


def _norm_body(od_ref, id_ref, ns_ref, nd_ref):
    o = od_ref[0] + od_ref[1]
    i = id_ref[0] + id_ref[1]
    ns_ref[...] = lax.rsqrt(jnp.maximum(o, 1.0))
    nd_ref[...] = lax.rsqrt(jnp.maximum(i, 1.0))


_BLKP = 1024

_norms = pl.pallas_call(
    _norm_body,
    grid=(NP // _BLKP,),
    in_specs=[pl.BlockSpec((NC, _BLKP, HW), lambda i: (0, i, 0)),
              pl.BlockSpec((NC, _BLKP, HW), lambda i: (0, i, 0))],
    out_specs=[pl.BlockSpec((_BLKP, HW), lambda i: (i, 0)),
               pl.BlockSpec((_BLKP, HW), lambda i: (i, 0))],
    out_shape=[jax.ShapeDtypeStruct((NP, HW), jnp.float32),
               jax.ShapeDtypeStruct((NP, HW), jnp.float32)],
)


def _dense_body(a_ref, w_ref, b_ref, nd_ref, hp_ref, o_ref):
    agg = a_ref[0] + a_ref[1]
    r = jnp.dot(agg, w_ref[...], preferred_element_type=jnp.float32)
    r = r * nd_ref[...][:, 0:1] + b_ref[...]
    o_ref[...] = jnp.maximum(r, 0.0) + hp_ref[...]


_dense = pl.pallas_call(
    _dense_body,
    grid=(N // _BLK,),
    in_specs=[pl.BlockSpec((NC, _BLK, D), lambda i: (0, i, 0)),
              pl.BlockSpec((D, D), lambda i: (0, 0)),
              pl.BlockSpec((1, D), lambda i: (0, 0)),
              pl.BlockSpec((_BLK, HW), lambda i: (i, 0)),
              pl.BlockSpec((_BLK, D), lambda i: (i, 0))],
    out_specs=pl.BlockSpec((_BLK, D), lambda i: (i, 0)),
    out_shape=jax.ShapeDtypeStruct((N, D), jnp.float32),
)


def kernel(x, edge_index, e_weight, W1, b1, W2, b2):
    src = edge_index[0].astype(jnp.int32)
    dst = edge_index[1].astype(jnp.int32)
    srcC = src.reshape(NW, NCH, CW)
    dstC = dst.reshape(NW, NCH, CW)
    ewC = e_weight.astype(jnp.float32).reshape(NW, NCH, CW)

    outdeg, indeg = _hist(srcC, dstC)
    ns16, nd16 = _norms(outdeg, indeg)
    ns = ns16[:, 0]

    h = x
    for (W, b) in ((W1, b1), (W2, b2)):
        aggp = _agg(h, srcC, dstC, ewC, ns)
        h = _dense(aggp, W, b.reshape(1, D), nd16, h)
    return h
